# Initial kernel scaffold; baseline (speedup 1.0000x reference)
import jax
import jax.numpy as jnp
from jax.experimental import pallas as pl


def kernel(nodes, history_uv, history_r, history_uvt, v2e_w, u2e_w, r2e_w, t2e_w, W1, b1, W2, b2, A1, a1b, A2, a2b, A3, a3b):
    raise NotImplementedError("write your pallas kernel here")


# speedup vs baseline: 5.1622x; 5.1622x over previous
import functools

import jax
import jax.numpy as jnp
from jax import lax
from jax.experimental import pallas as pl
from jax.experimental.pallas import tpu as pltpu
from jax.experimental.pallas import tpu_sc as plsc

B = 4096
L = 50
D = 64
CH = 128


def _sc_gather(table, idx2, nodes, utable):
    nchunks = idx2.shape[0]
    total = nchunks * CH
    info = plsc.get_sparse_core_info()
    nw = info.num_cores * info.num_subcores
    cpw = nchunks // nw
    npw = nodes.shape[0] // nw
    idx3 = idx2.reshape(nw, cpw, CH)

    @functools.partial(
        pl.kernel,
        mesh=plsc.VectorSubcoreMesh(core_axis_name="c", subcore_axis_name="s"),
        compiler_params=pltpu.CompilerParams(use_tc_tiling_on_sc=False),
        out_type=[
            jax.ShapeDtypeStruct((total, D), jnp.float32),
            jax.ShapeDtypeStruct((nodes.shape[0], D), jnp.float32),
        ],
        scratch_types=[
            pltpu.VMEM((cpw, CH), jnp.int32),
            pltpu.VMEM((CH, D), jnp.float32),
            pltpu.VMEM((npw,), jnp.int32),
            pltpu.VMEM((npw, D), jnp.float32),
            pltpu.SemaphoreType.DMA,
        ],
    )
    def k(table_hbm, idx_hbm, nodes_hbm, utable_hbm, out_hbm, uout_hbm,
          idx_v, rows_v, nidx_v, urows_v, sem):
        wid = lax.axis_index("s") * info.num_cores + lax.axis_index("c")
        pltpu.sync_copy(idx_hbm.at[wid], idx_v)

        def body(j, carry):
            pltpu.async_copy(table_hbm.at[idx_v.at[j]], rows_v, sem).wait()
            pltpu.sync_copy(rows_v, out_hbm.at[pl.ds((wid * cpw + j) * CH, CH)])
            return carry

        lax.fori_loop(0, cpw, body, 0)

        pltpu.sync_copy(nodes_hbm.at[pl.ds(wid * npw, npw)], nidx_v)
        pltpu.async_copy(utable_hbm.at[nidx_v], urows_v, sem).wait()
        pltpu.sync_copy(urows_v, uout_hbm.at[pl.ds(wid * npw, npw)])

    return k(table, idx3, nodes, utable)


def _tc_body(g_ref, hr_ref, u_ref, r2e_ref, W1_ref, b1_ref, W2_ref, b2_ref,
             A1_ref, a1b_ref, A2_ref, a2b_ref, A3_ref, out_ref):
    bb = out_ref.shape[0]
    dn = (((1,), (1,)), ((), ()))

    W1b = W1_ref[:, D:]
    r_tab = lax.dot_general(r2e_ref[...], W1b, dn,
                            preferred_element_type=jnp.float32) + b1_ref[...]
    hr = hr_ref[...][:, :, None]
    rc = jnp.zeros((L, bb, D), jnp.float32)
    for r in range(5):
        rc = rc + jnp.where(hr == r, 1.0, 0.0) * r_tab[r:r + 1][None]

    g = g_ref[...]
    gf = g.reshape(L * bb, D)
    W1a = W1_ref[:, :D]
    x1 = jnp.maximum(
        lax.dot_general(gf, W1a, dn, preferred_element_type=jnp.float32)
        + rc.reshape(L * bb, D), 0.0)
    oh = jnp.maximum(
        lax.dot_general(x1, W2_ref[...], dn, preferred_element_type=jnp.float32)
        + b2_ref[...], 0.0)

    A1a = A1_ref[:, :D]
    A1b = A1_ref[:, D:]
    p = lax.dot_general(oh, A1a, dn, preferred_element_type=jnp.float32)
    uc = lax.dot_general(u_ref[...], A1b, dn,
                         preferred_element_type=jnp.float32) + a1b_ref[...]
    a1 = jnp.maximum(p.reshape(L, bb, D) + uc[None], 0.0)
    a2 = jnp.maximum(
        lax.dot_general(a1.reshape(L * bb, D), A2_ref[...], dn,
                        preferred_element_type=jnp.float32) + a2b_ref[...], 0.0)
    t = jnp.sum(a2 * A3_ref[...], axis=-1, keepdims=True)
    t3 = t.reshape(L, bb, 1)
    m = jnp.max(t3, axis=0, keepdims=True)
    e = jnp.exp(t3 - m)
    w = e / jnp.sum(e, axis=0, keepdims=True)
    out_ref[...] = jnp.sum(oh.reshape(L, bb, D) * w, axis=0)


def _tc_compute(g3, hr3, urep, r2e_w, W1, b1, W2, b2, A1, a1b, A2, a2b, A3):
    bb = 128
    grid = B // bb
    full = lambda shape: pl.BlockSpec(shape, lambda i: tuple(0 for _ in shape))
    return pl.pallas_call(
        _tc_body,
        grid=(grid,),
        in_specs=[
            pl.BlockSpec((L, bb, D), lambda i: (0, i, 0)),
            pl.BlockSpec((L, bb), lambda i: (0, i)),
            pl.BlockSpec((bb, D), lambda i: (i, 0)),
            full((5, D)),
            full((D, 2 * D)),
            full((1, D)),
            full((D, D)),
            full((1, D)),
            full((D, 2 * D)),
            full((1, D)),
            full((D, D)),
            full((1, D)),
            full((1, D)),
        ],
        out_specs=pl.BlockSpec((bb, D), lambda i: (i, 0)),
        out_shape=jax.ShapeDtypeStruct((B, D), jnp.float32),
    )(g3, hr3, urep, r2e_w, W1, b1, W2, b2, A1, a1b, A2, a2b, A3)


def kernel(nodes, history_uv, history_r, history_uvt, v2e_w, u2e_w, r2e_w,
           t2e_w, W1, b1, W2, b2, A1, a1b, A2, a2b, A3, a3b):
    del history_uvt, t2e_w, a3b
    idx2 = history_uv.astype(jnp.int32).T.reshape(-1, CH)
    e_uv_flat, urep = _sc_gather(v2e_w, idx2, nodes.astype(jnp.int32), u2e_w)
    g3 = e_uv_flat.reshape(L, B, D)
    hr3 = history_r.astype(jnp.int32).T
    return _tc_compute(
        g3, hr3, urep, r2e_w, W1,
        b1.reshape(1, D), W2, b2.reshape(1, D),
        A1, a1b.reshape(1, D), A2, a2b.reshape(1, D), A3)


# speedup vs baseline: 7.8065x; 1.5122x over previous
import functools

import jax
import jax.numpy as jnp
from jax import lax
from jax.experimental import pallas as pl
from jax.experimental.pallas import tpu as pltpu
from jax.experimental.pallas import tpu_sc as plsc

B = 4096
L = 50
D = 64
CH = 128


def _sc_gather(table, hu_pages, nodes, utable):
    info = plsc.get_sparse_core_info()
    nw = info.num_cores * info.num_subcores
    npw = B // nw

    @functools.partial(
        pl.kernel,
        mesh=plsc.VectorSubcoreMesh(core_axis_name="c", subcore_axis_name="s"),
        compiler_params=pltpu.CompilerParams(use_tc_tiling_on_sc=False),
        out_type=[
            jax.ShapeDtypeStruct((L * B, D), jnp.float32),
            jax.ShapeDtypeStruct((B, D), jnp.float32),
        ],
        scratch_types=[
            pltpu.VMEM((L, CH), jnp.int32),
            pltpu.VMEM((CH, D), jnp.float32),
            pltpu.VMEM((CH, D), jnp.float32),
            pltpu.VMEM((npw,), jnp.int32),
            pltpu.VMEM((npw, D), jnp.float32),
            pltpu.SemaphoreType.DMA,
            pltpu.SemaphoreType.DMA,
        ],
    )
    def k(table_hbm, hu_hbm, nodes_hbm, utable_hbm, out_hbm, uout_hbm,
          idxt_v, rb0, rb1, nidx_v, urows_v, sem0, sem1):
        wid = lax.axis_index("s") * info.num_cores + lax.axis_index("c")
        b0 = wid * CH

        pltpu.sync_copy(hu_hbm.at[wid], idxt_v)

        pltpu.async_copy(table_hbm.at[idxt_v.at[0]], rb0, sem0)

        def gbody(j, carry):
            l0 = 2 * j
            pltpu.async_copy(table_hbm.at[idxt_v.at[l0 + 1]], rb1, sem1)
            pltpu.make_async_copy(table_hbm.at[idxt_v.at[l0]], rb0, sem0).wait()
            pltpu.sync_copy(rb0, out_hbm.at[pl.ds(l0 * B + b0, CH)])

            @pl.when(j < (L // 2 - 1))
            def _():
                pltpu.async_copy(table_hbm.at[idxt_v.at[l0 + 2]], rb0, sem0)

            pltpu.make_async_copy(table_hbm.at[idxt_v.at[l0 + 1]], rb1, sem1).wait()
            pltpu.sync_copy(rb1, out_hbm.at[pl.ds((l0 + 1) * B + b0, CH)])
            return carry

        lax.fori_loop(0, L // 2, gbody, 0)

        pltpu.sync_copy(nodes_hbm.at[pl.ds(wid * npw, npw)], nidx_v)
        pltpu.async_copy(utable_hbm.at[nidx_v], urows_v, sem0).wait()
        pltpu.sync_copy(urows_v, uout_hbm.at[pl.ds(wid * npw, npw)])

    return k(table, hu_pages, nodes, utable)


def _bd(x):
    z = jnp.zeros((D, D), jnp.float32)
    return jnp.concatenate(
        [jnp.concatenate([x, z], axis=1), jnp.concatenate([z, x], axis=1)],
        axis=0)


def _tc_body(g_ref, hrp_ref, u_ref, r2e_ref, W1_ref, b1_ref, W2_ref, b2_ref,
             A1_ref, a1b_ref, A2_ref, a2b_ref, A3_ref, out_ref):
    bp = out_ref.shape[0]
    r = L * bp
    dn = (((1,), (1,)), ((), ()))
    f32 = jnp.float32

    lane = lax.broadcasted_iota(jnp.int32, (2, 2 * D), 1)
    half = lax.broadcasted_iota(jnp.int32, (2, 2 * D), 0)
    Sm = jnp.where((lane < D) == (half == 0), 1.0, 0.0).astype(f32)

    r_tab = lax.dot_general(r2e_ref[...], W1_ref[:, D:], dn,
                            preferred_element_type=f32) + b1_ref[...]
    z5 = jnp.zeros((5, D), f32)
    Rt = jnp.concatenate(
        [jnp.concatenate([r_tab, z5], axis=1),
         jnp.concatenate([z5, r_tab], axis=1)], axis=0)
    col = lax.broadcasted_iota(jnp.int32, (2, 10), 1)
    hh = lax.broadcasted_iota(jnp.int32, (2, 10), 0)
    E = jnp.where((col < 5) == (hh == 0), 1.0, 0.0).astype(f32)
    rvals = jnp.where(col[:1] < 5, col[:1], col[:1] - 5).astype(f32)
    hf = hrp_ref[...].astype(f32).reshape(r, 2)
    hrep = lax.dot_general(hf, E, (((1,), (0,)), ((), ())),
                           preferred_element_type=f32)
    oh = jnp.where(hrep == rvals, 1.0, 0.0).astype(f32)
    rc = lax.dot_general(oh, Rt, (((1,), (0,)), ((), ())),
                         preferred_element_type=f32)

    gf = g_ref[...].reshape(r, 2 * D)
    x1 = jnp.maximum(
        lax.dot_general(gf, _bd(W1_ref[:, :D]), dn, preferred_element_type=f32)
        + rc, 0.0)
    b2d = jnp.concatenate([b2_ref[...], b2_ref[...]], axis=1)
    oh_ = jnp.maximum(
        lax.dot_general(x1, _bd(W2_ref[...]), dn, preferred_element_type=f32)
        + b2d, 0.0)

    p = lax.dot_general(oh_, _bd(A1_ref[:, :D]), dn, preferred_element_type=f32)
    a1bd = jnp.concatenate([a1b_ref[...], a1b_ref[...]], axis=1)
    uc = lax.dot_general(u_ref[...], _bd(A1_ref[:, D:]), dn,
                         preferred_element_type=f32) + a1bd
    a1 = jnp.maximum(p.reshape(L, bp, 2 * D) + uc[None], 0.0)
    a2bd = jnp.concatenate([a2b_ref[...], a2b_ref[...]], axis=1)
    a2 = jnp.maximum(
        lax.dot_general(a1.reshape(r, 2 * D), _bd(A2_ref[...]), dn,
                        preferred_element_type=f32) + a2bd, 0.0)

    A3d = jnp.concatenate([A3_ref[...], A3_ref[...]], axis=1)
    t2 = lax.dot_general(a2 * A3d, Sm, dn, preferred_element_type=f32)
    t3 = t2.reshape(L, bp, 2)
    m = jnp.max(t3, axis=0, keepdims=True)
    e = jnp.exp(t3 - m)
    w3 = e / jnp.sum(e, axis=0, keepdims=True)
    wf = lax.dot_general(w3.reshape(r, 2), Sm, (((1,), (0,)), ((), ())),
                         preferred_element_type=f32)
    out_ref[...] = jnp.sum((oh_ * wf).reshape(L, bp, 2 * D), axis=0)


def _tc_compute(g2, hrp, urep2, r2e_w, W1, b1, W2, b2, A1, a1b, A2, a2b, A3):
    bp = 128
    grid = (B // 2) // bp
    full = lambda shape: pl.BlockSpec(shape, lambda i: tuple(0 for _ in shape))
    return pl.pallas_call(
        _tc_body,
        grid=(grid,),
        in_specs=[
            pl.BlockSpec((L, bp, 2 * D), lambda i: (0, i, 0)),
            pl.BlockSpec((L, bp, 2), lambda i: (0, i, 0)),
            pl.BlockSpec((bp, 2 * D), lambda i: (i, 0)),
            full((5, D)),
            full((D, 2 * D)),
            full((1, D)),
            full((D, D)),
            full((1, D)),
            full((D, 2 * D)),
            full((1, D)),
            full((D, D)),
            full((1, D)),
            full((1, D)),
        ],
        out_specs=pl.BlockSpec((bp, 2 * D), lambda i: (i, 0)),
        out_shape=jax.ShapeDtypeStruct((B // 2, 2 * D), jnp.float32),
    )(g2, hrp, urep2, r2e_w, W1, b1, W2, b2, A1, a1b, A2, a2b, A3)


def kernel(nodes, history_uv, history_r, history_uvt, v2e_w, u2e_w, r2e_w,
           t2e_w, W1, b1, W2, b2, A1, a1b, A2, a2b, A3, a3b):
    del history_uvt, t2e_w, a3b
    info = plsc.get_sparse_core_info()
    nw = info.num_cores * info.num_subcores
    hu_pages = jnp.transpose(history_uv.astype(jnp.int32)) \
        .reshape(L, nw, CH).transpose(1, 0, 2)
    e_uv_flat, urep = _sc_gather(
        v2e_w, hu_pages, nodes.astype(jnp.int32), u2e_w)
    g2 = e_uv_flat.reshape(L, B // 2, 2 * D)
    hrp = jnp.transpose(history_r.astype(jnp.int32)).reshape(L, B // 2, 2)
    urep2 = urep.reshape(B // 2, 2 * D)
    out2 = _tc_compute(
        g2, hrp, urep2, r2e_w, W1,
        b1.reshape(1, D), W2, b2.reshape(1, D),
        A1, a1b.reshape(1, D), A2, a2b.reshape(1, D), A3)
    return out2.reshape(B, D)
